# Initial kernel scaffold; baseline (speedup 1.0000x reference)
import jax
import jax.numpy as jnp
from jax.experimental import pallas as pl


def kernel(x, edge_index, edge_attr, W1, b1, att_l0, att_r0, att_l1, att_r1, W2, b2):
    raise NotImplementedError("write your pallas kernel here")


# speedup vs baseline: 9.0660x; 9.0660x over previous
import functools

import jax
import jax.numpy as jnp
from jax import lax
from jax.experimental import pallas as pl
from jax.experimental.pallas import tpu as pltpu
from jax.experimental.pallas import tpu_sc as plsc

N = 10000
E = 320000
D = 128
H = 64
C = 16
EPS = 0.1

NC = 2
NS = 16
L = 16
NW = NC * NS

ROW = 128
SUB = 3
CHUNK_E = ROW * SUB
CPW = 28
NQUAD = CPW // 4
E2 = NW * CPW * CHUNK_E
NPAD = 10240
ROWS_PER_TILE = NPAD // NS

TC_BLK = 1000
TC_GRID = N // TC_BLK


def _tc_pre_body(x_ref, w1_ref, b1_ref, attl_ref, attr_ref, h_ref, al_ref,
                 ar_ref):
    h = jnp.dot(x_ref[...], w1_ref[...], preferred_element_type=jnp.float32)
    h = jnp.maximum(h + b1_ref[...], 0.0)
    h_ref[...] = h
    al_ref[...] = jnp.dot(h, attl_ref[...], preferred_element_type=jnp.float32)
    ar_ref[...] = jnp.dot(h, attr_ref[...], preferred_element_type=jnp.float32)


def _tc_mid_body(agg_ref, x0_ref, attl_ref, attr_ref, h_ref, al_ref, ar_ref):
    h = agg_ref[0] + agg_ref[1] + EPS * x0_ref[...]
    h_ref[...] = h
    al_ref[...] = jnp.dot(h, attl_ref[...], preferred_element_type=jnp.float32)
    ar_ref[...] = jnp.dot(h, attr_ref[...], preferred_element_type=jnp.float32)


def _tc_post_body(agg_ref, x0_ref, w2_ref, b2_ref, out_ref):
    h = agg_ref[0] + agg_ref[1] + EPS * x0_ref[...]
    z = jnp.dot(h, w2_ref[...], preferred_element_type=jnp.float32)
    z = z + b2_ref[...]
    m = jnp.max(z, axis=1, keepdims=True)
    lse = jnp.log(jnp.sum(jnp.exp(z - m), axis=1, keepdims=True)) + m
    out_ref[...] = z - lse


def _tc_pre(x, W1, b1, attl, attr):
    return pl.pallas_call(
        _tc_pre_body,
        grid=(TC_GRID,),
        in_specs=[
            pl.BlockSpec((TC_BLK, D), lambda i: (i, 0)),
            pl.BlockSpec((D, H), lambda i: (0, 0)),
            pl.BlockSpec((1, H), lambda i: (0, 0)),
            pl.BlockSpec((H, 1), lambda i: (0, 0)),
            pl.BlockSpec((H, 1), lambda i: (0, 0)),
        ],
        out_specs=[
            pl.BlockSpec((TC_BLK, H), lambda i: (i, 0)),
            pl.BlockSpec((TC_BLK, 1), lambda i: (i, 0)),
            pl.BlockSpec((TC_BLK, 1), lambda i: (i, 0)),
        ],
        out_shape=[
            jax.ShapeDtypeStruct((N, H), jnp.float32),
            jax.ShapeDtypeStruct((N, 1), jnp.float32),
            jax.ShapeDtypeStruct((N, 1), jnp.float32),
        ],
    )(x, W1, b1.reshape(1, H), attl.reshape(H, 1), attr.reshape(H, 1))


def _tc_mid(agg, x0, attl, attr):
    return pl.pallas_call(
        _tc_mid_body,
        grid=(TC_GRID,),
        in_specs=[
            pl.BlockSpec((2, TC_BLK, H), lambda i: (0, i, 0)),
            pl.BlockSpec((TC_BLK, H), lambda i: (i, 0)),
            pl.BlockSpec((H, 1), lambda i: (0, 0)),
            pl.BlockSpec((H, 1), lambda i: (0, 0)),
        ],
        out_specs=[
            pl.BlockSpec((TC_BLK, H), lambda i: (i, 0)),
            pl.BlockSpec((TC_BLK, 1), lambda i: (i, 0)),
            pl.BlockSpec((TC_BLK, 1), lambda i: (i, 0)),
        ],
        out_shape=[
            jax.ShapeDtypeStruct((N, H), jnp.float32),
            jax.ShapeDtypeStruct((N, 1), jnp.float32),
            jax.ShapeDtypeStruct((N, 1), jnp.float32),
        ],
    )(agg, x0, attl.reshape(H, 1), attr.reshape(H, 1))


def _tc_post(agg, x0, W2, b2):
    return pl.pallas_call(
        _tc_post_body,
        grid=(TC_GRID,),
        in_specs=[
            pl.BlockSpec((2, TC_BLK, H), lambda i: (0, i, 0)),
            pl.BlockSpec((TC_BLK, H), lambda i: (i, 0)),
            pl.BlockSpec((H, C), lambda i: (0, 0)),
            pl.BlockSpec((1, C), lambda i: (0, 0)),
        ],
        out_specs=pl.BlockSpec((TC_BLK, C), lambda i: (i, 0)),
        out_shape=jax.ShapeDtypeStruct((N, C), jnp.float32),
    )(agg, x0, W2, b2.reshape(1, C))


def _sc_layer_body(h_hbm, src_hbm, dst_hbm, ea_hbm, al_hbm, ar_hbm, zeros_hbm,
                   out_hbm, agg_s, al_v, ar_v, src_v, dst_v, ea_v, coef_v,
                   rows_v, gsem, isem, ssem0, ssem1):
    c = lax.axis_index("c")
    s = lax.axis_index("s")
    w = c * NS + s
    lo = w * CPW

    def issue_idx(k, q):
        off = k * CHUNK_E
        pltpu.async_copy(src_hbm.at[pl.ds(off, CHUNK_E)], src_v.at[q], isem)
        pltpu.async_copy(ea_hbm.at[pl.ds(off, CHUNK_E)], ea_v.at[q], isem)
        for j in range(SUB):
            pltpu.async_copy(dst_hbm.at[pl.ds(off + j * ROW, ROW)],
                             dst_v.at[q].at[j], isem)

    def wait_idx(q):
        pltpu.make_async_copy(src_hbm.at[pl.ds(0, CHUNK_E)], src_v.at[q],
                              isem).wait()
        pltpu.make_async_copy(ea_hbm.at[pl.ds(0, CHUNK_E)], ea_v.at[q],
                              isem).wait()
        for j in range(SUB):
            pltpu.make_async_copy(dst_hbm.at[pl.ds(0, ROW)],
                                  dst_v.at[q].at[j], isem).wait()

    def issue_gathers(p, q):
        for j in range(SUB):
            pltpu.async_copy(h_hbm.at[src_v.at[q].at[pl.ds(j * ROW, ROW)]],
                             rows_v.at[p].at[j], gsem)

    def wait_gathers(p, q):
        for j in range(SUB):
            pltpu.make_async_copy(h_hbm.at[src_v.at[q].at[pl.ds(0, ROW)]],
                                  rows_v.at[p].at[j], gsem).wait()

    def issue_scatters(p, q):
        sem = ssem0 if p == 0 else ssem1
        for j in range(SUB):
            pltpu.async_copy(rows_v.at[p].at[j], agg_s.at[dst_v.at[q].at[j]],
                             sem, add=True)

    def wait_scatters(p, q):
        sem = ssem0 if p == 0 else ssem1
        for j in range(SUB):
            pltpu.make_async_copy(rows_v.at[p].at[j],
                                  agg_s.at[dst_v.at[q].at[j]], sem).wait()

    def coef(q):
        for g in range(CHUNK_E // L):
            sl = pl.ds(g * L, L)
            sidx = src_v[q, sl]
            didx = dst_v[q, g // (ROW // L), pl.ds((g % (ROW // L)) * L, L)]
            z = plsc.load_gather(al_v, [sidx]) + plsc.load_gather(ar_v, [didx])
            az = jnp.abs(z)
            t = 1.0 - 2.0 / (jnp.exp(2.0 * az) + 1.0)
            t = jnp.where(z < 0.0, -t, t)
            coef_v[sl] = t * ea_v[q, sl]

    def scale(p):
        for j in range(SUB):
            def scale_grp(g2, _, j=j):
                cvec = coef_v[pl.ds(j * ROW + g2 * L, L)]
                for i2 in range(L):
                    cf = cvec[i2]
                    r = g2 * L + i2
                    for d in range(H // L):
                        dl = pl.ds(d * L, L)
                        rows_v[p, j, r, dl] = rows_v[p, j, r, dl] * cf
                return 0
            lax.fori_loop(0, ROW // L, scale_grp, 0)

    pltpu.sync_copy(al_hbm, al_v)
    pltpu.sync_copy(ar_hbm, ar_v)
    zslc = pl.ds(s * ROWS_PER_TILE, ROWS_PER_TILE)
    pltpu.sync_copy(zeros_hbm, agg_s.at[zslc])
    issue_idx(lo, 0)
    plsc.subcore_barrier()

    def quad(i, carry):
        for q in range(4):
            p = q % 2
            k = lo + 4 * i + q
            if q < 2:
                @pl.when(i > 0)
                def _():
                    wait_scatters(p, q + 2)
            else:
                wait_scatters(p, q - 2)
            wait_idx(q)
            if q == 3:
                @pl.when(i < NQUAD - 1)
                def _():
                    issue_idx(k + 1, 0)
            else:
                issue_idx(k + 1, q + 1)
            issue_gathers(p, q)
            coef(q)
            wait_gathers(p, q)
            scale(p)
            issue_scatters(p, q)
        return carry

    lax.fori_loop(0, NQUAD, quad, 0)

    wait_scatters(0, 2)
    wait_scatters(1, 3)
    plsc.subcore_barrier()
    pltpu.sync_copy(agg_s.at[zslc], out_hbm.at[c].at[zslc])


def _sc_layer(h, src_p, dst_p, ea_p, al, ar, zeros_blk):
    mesh = plsc.VectorSubcoreMesh(core_axis_name="c", subcore_axis_name="s")
    fn = pl.kernel(
        _sc_layer_body,
        out_type=jax.ShapeDtypeStruct((NC, NPAD, H), jnp.float32),
        mesh=mesh,
        compiler_params=pltpu.CompilerParams(
            needs_layout_passes=False, use_tc_tiling_on_sc=False),
        scratch_types=[
            pltpu.VMEM_SHARED((NPAD, H), jnp.float32),
            pltpu.VMEM((N,), jnp.float32),
            pltpu.VMEM((N,), jnp.float32),
            pltpu.VMEM((4, CHUNK_E), jnp.int32),
            pltpu.VMEM((4, SUB, ROW), jnp.int32),
            pltpu.VMEM((4, CHUNK_E), jnp.float32),
            pltpu.VMEM((CHUNK_E,), jnp.float32),
            pltpu.VMEM((2, SUB, ROW, H), jnp.float32),
            pltpu.SemaphoreType.DMA,
            pltpu.SemaphoreType.DMA,
            pltpu.SemaphoreType.DMA,
            pltpu.SemaphoreType.DMA,
        ],
    )
    return fn(h, src_p, dst_p, ea_p, al, ar, zeros_blk)


def kernel(x, edge_index, edge_attr, W1, b1, att_l0, att_r0, att_l1, att_r1,
           W2, b2):
    pad = E2 - E
    src_p = jnp.concatenate([edge_index[0], jnp.zeros((pad,), jnp.int32)])
    dst_p = jnp.concatenate([edge_index[1], jnp.zeros((pad,), jnp.int32)])
    ea_p = jnp.concatenate([edge_attr, jnp.zeros((pad,), jnp.float32)])
    zeros_blk = jnp.zeros((ROWS_PER_TILE, H), jnp.float32)

    h, al, ar = _tc_pre(x, W1, b1, att_l0, att_r0)
    x0 = h

    agg = _sc_layer(h, src_p, dst_p, ea_p, al.reshape(N), ar.reshape(N),
                    zeros_blk)
    h, al, ar = _tc_mid(agg, x0, att_l1, att_r1)

    agg = _sc_layer(h, src_p, dst_p, ea_p, al.reshape(N), ar.reshape(N),
                    zeros_blk)
    return _tc_post(agg, x0, W2, b2)
